# Initial kernel scaffold; baseline (speedup 1.0000x reference)
import jax
import jax.numpy as jnp
from jax.experimental import pallas as pl


def kernel(x, pe, Wk, Wv, W1, gamma, beta, W2):
    raise NotImplementedError("write your pallas kernel here")


# speedup vs baseline: 1.7316x; 1.7316x over previous
import jax
import jax.numpy as jnp
from jax.experimental import pallas as pl
from jax.experimental.pallas import tpu as pltpu

_LN_EPS = 1e-3
_S1 = 4096
_S2 = 4096


def _pool_kernel(x_ref, pe_ref, wv_ref, wk_ref, w1_ref, g_ref, b_ref, w2_ref,
                 out_ref, acc_ref, m_ref, l_ref, peacc_ref):
    t = pl.program_id(1)
    nt = pl.num_programs(1)

    xb = x_ref[0]
    pe_row = pe_ref[...]
    wv_row = wv_ref[...]

    logits = jax.lax.dot_general(
        wv_row, xb, (((1,), (1,)), ((), ())),
        preferred_element_type=jnp.float32)
    logits = logits + pe_row * jnp.sum(wv_row, axis=-1, keepdims=True)

    tmax = jnp.max(logits)

    @pl.when(t == 0)
    def _init():
        p = jnp.exp(logits - tmax)
        m_ref[0, 0] = tmax
        l_ref[0, 0] = jnp.sum(p)
        peacc_ref[0, 0] = jnp.sum(p * pe_row)
        acc_ref[...] = jnp.dot(p, xb, preferred_element_type=jnp.float32)

    @pl.when(t > 0)
    def _update():
        m_old = m_ref[0, 0]
        m_new = jnp.maximum(m_old, tmax)
        corr = jnp.exp(m_old - m_new)
        p = jnp.exp(logits - m_new)
        m_ref[0, 0] = m_new
        l_ref[0, 0] = l_ref[0, 0] * corr + jnp.sum(p)
        peacc_ref[0, 0] = peacc_ref[0, 0] * corr + jnp.sum(p * pe_row)
        acc_ref[...] = acc_ref[...] * corr + jnp.dot(
            p, xb, preferred_element_type=jnp.float32)

    @pl.when(t == nt - 1)
    def _epilogue():
        inv_l = 1.0 / l_ref[0, 0]
        pooled = (acc_ref[...] + peacc_ref[0, 0]) * inv_l
        context = jnp.dot(pooled, wk_ref[...],
                          preferred_element_type=jnp.float32)
        h = jnp.dot(context, w1_ref[...],
                    preferred_element_type=jnp.float32)
        mu = jnp.mean(h, axis=-1, keepdims=True)
        var = jnp.mean(jnp.square(h - mu), axis=-1, keepdims=True)
        h = (h - mu) * jax.lax.rsqrt(var + _LN_EPS) * g_ref[...] + b_ref[...]
        h = jnp.clip(h, 0.0, 6.0)
        out_ref[0] = jnp.dot(h, w2_ref[...],
                             preferred_element_type=jnp.float32)


def _apply_kernel(x_ref, pe_ref, ca_ref, o_ref):
    o_ref[0] = x_ref[0] + pe_ref[...] + ca_ref[0]


def kernel(x, pe, Wk, Wv, W1, gamma, beta, W2):
    B, H, W, C = x.shape
    HW = H * W
    MID = W1.shape[1]

    xf = x.reshape(B, HW, C)
    pe_row = pe.reshape(1, HW)
    pe_col = pe.reshape(HW, 1)
    wv_row = Wv.reshape(1, C)
    g_row = gamma.reshape(1, MID)
    b_row = beta.reshape(1, MID)

    nt1 = HW // _S1
    ca = pl.pallas_call(
        _pool_kernel,
        grid=(B, nt1),
        in_specs=[
            pl.BlockSpec((1, _S1, C), lambda b, t: (b, t, 0)),
            pl.BlockSpec((1, _S1), lambda b, t: (0, t)),
            pl.BlockSpec((1, C), lambda b, t: (0, 0)),
            pl.BlockSpec((C, C), lambda b, t: (0, 0)),
            pl.BlockSpec((C, MID), lambda b, t: (0, 0)),
            pl.BlockSpec((1, MID), lambda b, t: (0, 0)),
            pl.BlockSpec((1, MID), lambda b, t: (0, 0)),
            pl.BlockSpec((MID, C), lambda b, t: (0, 0)),
        ],
        out_specs=pl.BlockSpec((1, 1, C), lambda b, t: (b, 0, 0)),
        out_shape=jax.ShapeDtypeStruct((B, 1, C), jnp.float32),
        scratch_shapes=[
            pltpu.VMEM((1, C), jnp.float32),
            pltpu.SMEM((1, 1), jnp.float32),
            pltpu.SMEM((1, 1), jnp.float32),
            pltpu.SMEM((1, 1), jnp.float32),
        ],
        compiler_params=pltpu.CompilerParams(
            dimension_semantics=("parallel", "arbitrary")),
    )(xf, pe_row, wv_row, Wk, W1, g_row, b_row, W2)

    nt2 = HW // _S2
    out = pl.pallas_call(
        _apply_kernel,
        grid=(B, nt2),
        in_specs=[
            pl.BlockSpec((1, _S2, C), lambda b, t: (b, t, 0)),
            pl.BlockSpec((_S2, 1), lambda b, t: (t, 0)),
            pl.BlockSpec((1, 1, C), lambda b, t: (b, 0, 0)),
        ],
        out_specs=pl.BlockSpec((1, _S2, C), lambda b, t: (b, t, 0)),
        out_shape=jax.ShapeDtypeStruct((B, HW, C), jnp.float32),
        compiler_params=pltpu.CompilerParams(
            dimension_semantics=("parallel", "arbitrary")),
    )(xf, pe_col, ca)

    return out.reshape(B, H, W, C)


# speedup vs baseline: 1.8460x; 1.0660x over previous
import jax
import jax.numpy as jnp
from jax.experimental import pallas as pl
from jax.experimental.pallas import tpu as pltpu

_LN_EPS = 1e-3
_S1 = 8192
_S2 = 8192


def _pool_kernel(x_ref, pe_ref, wv_ref, wk_ref, w1_ref, g_ref, b_ref, w2_ref,
                 out_ref, acc_ref, m_ref, l_ref, peacc_ref):
    t = pl.program_id(1)
    nt = pl.num_programs(1)

    xb = x_ref[0]
    pe_row = pe_ref[...]
    wv_row = wv_ref[...]

    logits = jax.lax.dot_general(
        wv_row, xb, (((1,), (1,)), ((), ())),
        preferred_element_type=jnp.float32)
    logits = logits + pe_row * jnp.sum(wv_row, axis=-1, keepdims=True)

    tmax = jnp.max(logits)

    @pl.when(t == 0)
    def _init():
        p = jnp.exp(logits - tmax)
        m_ref[0, 0] = tmax
        l_ref[0, 0] = jnp.sum(p)
        peacc_ref[0, 0] = jnp.sum(p * pe_row)
        acc_ref[...] = jnp.dot(p, xb, preferred_element_type=jnp.float32)

    @pl.when(t > 0)
    def _update():
        m_old = m_ref[0, 0]
        m_new = jnp.maximum(m_old, tmax)
        corr = jnp.exp(m_old - m_new)
        p = jnp.exp(logits - m_new)
        m_ref[0, 0] = m_new
        l_ref[0, 0] = l_ref[0, 0] * corr + jnp.sum(p)
        peacc_ref[0, 0] = peacc_ref[0, 0] * corr + jnp.sum(p * pe_row)
        acc_ref[...] = acc_ref[...] * corr + jnp.dot(
            p, xb, preferred_element_type=jnp.float32)

    @pl.when(t == nt - 1)
    def _epilogue():
        inv_l = 1.0 / l_ref[0, 0]
        pooled = (acc_ref[...] + peacc_ref[0, 0]) * inv_l
        context = jnp.dot(pooled, wk_ref[...],
                          preferred_element_type=jnp.float32)
        h = jnp.dot(context, w1_ref[...],
                    preferred_element_type=jnp.float32)
        mu = jnp.mean(h, axis=-1, keepdims=True)
        var = jnp.mean(jnp.square(h - mu), axis=-1, keepdims=True)
        h = (h - mu) * jax.lax.rsqrt(var + _LN_EPS) * g_ref[...] + b_ref[...]
        h = jnp.clip(h, 0.0, 6.0)
        out_ref[0] = jnp.dot(h, w2_ref[...],
                             preferred_element_type=jnp.float32)


def _apply_kernel(x_ref, pe_ref, ca_ref, o_ref):
    o_ref[0] = x_ref[0] + pe_ref[...] + ca_ref[0]


def kernel(x, pe, Wk, Wv, W1, gamma, beta, W2):
    B, H, W, C = x.shape
    HW = H * W
    MID = W1.shape[1]

    xf = x.reshape(B, HW, C)
    pe_row = pe.reshape(1, HW)
    pe_col = pe.reshape(HW, 1)
    wv_row = Wv.reshape(1, C)
    g_row = gamma.reshape(1, MID)
    b_row = beta.reshape(1, MID)

    nt1 = HW // _S1
    ca = pl.pallas_call(
        _pool_kernel,
        grid=(B, nt1),
        in_specs=[
            pl.BlockSpec((1, _S1, C), lambda b, t: (b, t, 0)),
            pl.BlockSpec((1, _S1), lambda b, t: (0, t)),
            pl.BlockSpec((1, C), lambda b, t: (0, 0)),
            pl.BlockSpec((C, C), lambda b, t: (0, 0)),
            pl.BlockSpec((C, MID), lambda b, t: (0, 0)),
            pl.BlockSpec((1, MID), lambda b, t: (0, 0)),
            pl.BlockSpec((1, MID), lambda b, t: (0, 0)),
            pl.BlockSpec((MID, C), lambda b, t: (0, 0)),
        ],
        out_specs=pl.BlockSpec((1, 1, C), lambda b, t: (b, 0, 0)),
        out_shape=jax.ShapeDtypeStruct((B, 1, C), jnp.float32),
        scratch_shapes=[
            pltpu.VMEM((1, C), jnp.float32),
            pltpu.SMEM((1, 1), jnp.float32),
            pltpu.SMEM((1, 1), jnp.float32),
            pltpu.SMEM((1, 1), jnp.float32),
        ],
        compiler_params=pltpu.CompilerParams(
            dimension_semantics=("parallel", "arbitrary")),
    )(xf, pe_row, wv_row, Wk, W1, g_row, b_row, W2)

    nt2 = HW // _S2
    out = pl.pallas_call(
        _apply_kernel,
        grid=(B, nt2),
        in_specs=[
            pl.BlockSpec((1, _S2, C), lambda b, t: (b, t, 0)),
            pl.BlockSpec((_S2, 1), lambda b, t: (t, 0)),
            pl.BlockSpec((1, 1, C), lambda b, t: (b, 0, 0)),
        ],
        out_specs=pl.BlockSpec((1, _S2, C), lambda b, t: (b, t, 0)),
        out_shape=jax.ShapeDtypeStruct((B, HW, C), jnp.float32),
        compiler_params=pltpu.CompilerParams(
            dimension_semantics=("parallel", "parallel")),
    )(xf, pe_col, ca)

    return out.reshape(B, H, W, C)


# speedup vs baseline: 1.8466x; 1.0003x over previous
import jax
import jax.numpy as jnp
from jax.experimental import pallas as pl
from jax.experimental.pallas import tpu as pltpu

_LN_EPS = 1e-3
_S1 = 8192
_S2 = 8192


def _pool_kernel(x_ref, pe_ref, wv_ref, wk_ref, w1_ref, g_ref, b_ref, w2_ref,
                 out_ref, acc_ref, m_ref, l_ref, peacc_ref):
    t = pl.program_id(1)
    nt = pl.num_programs(1)

    xb = x_ref[0]
    xb16 = xb.astype(jnp.bfloat16)
    pe_row = pe_ref[...]
    wv_row = wv_ref[...]

    logits = jax.lax.dot_general(
        wv_row.astype(jnp.bfloat16), xb16, (((1,), (1,)), ((), ())),
        preferred_element_type=jnp.float32)
    logits = logits + pe_row * jnp.sum(wv_row, axis=-1, keepdims=True)

    tmax = jnp.max(logits)

    @pl.when(t == 0)
    def _init():
        p = jnp.exp(logits - tmax)
        m_ref[0, 0] = tmax
        l_ref[0, 0] = jnp.sum(p)
        peacc_ref[0, 0] = jnp.sum(p * pe_row)
        acc_ref[...] = jnp.dot(p.astype(jnp.bfloat16), xb16,
                               preferred_element_type=jnp.float32)

    @pl.when(t > 0)
    def _update():
        m_old = m_ref[0, 0]
        m_new = jnp.maximum(m_old, tmax)
        corr = jnp.exp(m_old - m_new)
        p = jnp.exp(logits - m_new)
        m_ref[0, 0] = m_new
        l_ref[0, 0] = l_ref[0, 0] * corr + jnp.sum(p)
        peacc_ref[0, 0] = peacc_ref[0, 0] * corr + jnp.sum(p * pe_row)
        acc_ref[...] = acc_ref[...] * corr + jnp.dot(
            p.astype(jnp.bfloat16), xb16, preferred_element_type=jnp.float32)

    @pl.when(t == nt - 1)
    def _epilogue():
        inv_l = 1.0 / l_ref[0, 0]
        pooled = (acc_ref[...] + peacc_ref[0, 0]) * inv_l
        context = jnp.dot(pooled, wk_ref[...],
                          preferred_element_type=jnp.float32)
        h = jnp.dot(context, w1_ref[...],
                    preferred_element_type=jnp.float32)
        mu = jnp.mean(h, axis=-1, keepdims=True)
        var = jnp.mean(jnp.square(h - mu), axis=-1, keepdims=True)
        h = (h - mu) * jax.lax.rsqrt(var + _LN_EPS) * g_ref[...] + b_ref[...]
        h = jnp.clip(h, 0.0, 6.0)
        out_ref[0] = jnp.dot(h, w2_ref[...],
                             preferred_element_type=jnp.float32)


def _apply_kernel(x_ref, pe_ref, ca_ref, o_ref):
    o_ref[0] = x_ref[0] + pe_ref[...] + ca_ref[0]


def kernel(x, pe, Wk, Wv, W1, gamma, beta, W2):
    B, H, W, C = x.shape
    HW = H * W
    MID = W1.shape[1]

    xf = x.reshape(B, HW, C)
    pe_row = pe.reshape(1, HW)
    pe_col = pe.reshape(HW, 1)
    wv_row = Wv.reshape(1, C)
    g_row = gamma.reshape(1, MID)
    b_row = beta.reshape(1, MID)

    nt1 = HW // _S1
    ca = pl.pallas_call(
        _pool_kernel,
        grid=(B, nt1),
        in_specs=[
            pl.BlockSpec((1, _S1, C), lambda b, t: (b, t, 0)),
            pl.BlockSpec((1, _S1), lambda b, t: (0, t)),
            pl.BlockSpec((1, C), lambda b, t: (0, 0)),
            pl.BlockSpec((C, C), lambda b, t: (0, 0)),
            pl.BlockSpec((C, MID), lambda b, t: (0, 0)),
            pl.BlockSpec((1, MID), lambda b, t: (0, 0)),
            pl.BlockSpec((1, MID), lambda b, t: (0, 0)),
            pl.BlockSpec((MID, C), lambda b, t: (0, 0)),
        ],
        out_specs=pl.BlockSpec((1, 1, C), lambda b, t: (b, 0, 0)),
        out_shape=jax.ShapeDtypeStruct((B, 1, C), jnp.float32),
        scratch_shapes=[
            pltpu.VMEM((1, C), jnp.float32),
            pltpu.SMEM((1, 1), jnp.float32),
            pltpu.SMEM((1, 1), jnp.float32),
            pltpu.SMEM((1, 1), jnp.float32),
        ],
        compiler_params=pltpu.CompilerParams(
            dimension_semantics=("parallel", "arbitrary")),
    )(xf, pe_row, wv_row, Wk, W1, g_row, b_row, W2)

    nt2 = HW // _S2
    out = pl.pallas_call(
        _apply_kernel,
        grid=(B, nt2),
        in_specs=[
            pl.BlockSpec((1, _S2, C), lambda b, t: (b, t, 0)),
            pl.BlockSpec((_S2, 1), lambda b, t: (t, 0)),
            pl.BlockSpec((1, 1, C), lambda b, t: (b, 0, 0)),
        ],
        out_specs=pl.BlockSpec((1, _S2, C), lambda b, t: (b, t, 0)),
        out_shape=jax.ShapeDtypeStruct((B, HW, C), jnp.float32),
        compiler_params=pltpu.CompilerParams(
            dimension_semantics=("parallel", "parallel")),
    )(xf, pe_col, ca)

    return out.reshape(B, H, W, C)


# speedup vs baseline: 1.9026x; 1.0303x over previous
import jax
import jax.numpy as jnp
from jax.experimental import pallas as pl
from jax.experimental.pallas import tpu as pltpu

_LN_EPS = 1e-3
_S1 = 16384
_S2 = 8192


def _pool_kernel(x_ref, pe_ref, wv_ref, wk_ref, w1_ref, g_ref, b_ref, w2_ref,
                 out_ref, acc_ref, m_ref, l_ref, peacc_ref):
    t = pl.program_id(1)
    nt = pl.num_programs(1)

    xb = x_ref[0]
    pe_row = pe_ref[...]
    wv_row = wv_ref[...]

    logits = jax.lax.dot_general(
        wv_row, xb, (((1,), (1,)), ((), ())),
        preferred_element_type=jnp.float32)
    logits = logits + pe_row * jnp.sum(wv_row, axis=-1, keepdims=True)

    tmax = jnp.max(logits)

    @pl.when(t == 0)
    def _init():
        p = jnp.exp(logits - tmax)
        m_ref[0, 0] = tmax
        l_ref[0, 0] = jnp.sum(p)
        peacc_ref[0, 0] = jnp.sum(p * pe_row)
        acc_ref[...] = jnp.dot(p, xb, preferred_element_type=jnp.float32)

    @pl.when(t > 0)
    def _update():
        m_old = m_ref[0, 0]
        m_new = jnp.maximum(m_old, tmax)
        corr = jnp.exp(m_old - m_new)
        p = jnp.exp(logits - m_new)
        m_ref[0, 0] = m_new
        l_ref[0, 0] = l_ref[0, 0] * corr + jnp.sum(p)
        peacc_ref[0, 0] = peacc_ref[0, 0] * corr + jnp.sum(p * pe_row)
        acc_ref[...] = acc_ref[...] * corr + jnp.dot(
            p, xb, preferred_element_type=jnp.float32)

    @pl.when(t == nt - 1)
    def _epilogue():
        inv_l = 1.0 / l_ref[0, 0]
        pooled = (acc_ref[...] + peacc_ref[0, 0]) * inv_l
        context = jnp.dot(pooled, wk_ref[...],
                          preferred_element_type=jnp.float32)
        h = jnp.dot(context, w1_ref[...],
                    preferred_element_type=jnp.float32)
        mu = jnp.mean(h, axis=-1, keepdims=True)
        var = jnp.mean(jnp.square(h - mu), axis=-1, keepdims=True)
        h = (h - mu) * jax.lax.rsqrt(var + _LN_EPS) * g_ref[...] + b_ref[...]
        h = jnp.clip(h, 0.0, 6.0)
        out_ref[0] = jnp.dot(h, w2_ref[...],
                             preferred_element_type=jnp.float32)


def _apply_kernel(x_ref, pe_ref, ca_ref, o_ref):
    o_ref[0] = x_ref[0] + pe_ref[...] + ca_ref[0]


def kernel(x, pe, Wk, Wv, W1, gamma, beta, W2):
    B, H, W, C = x.shape
    HW = H * W
    MID = W1.shape[1]

    xf = x.reshape(B, HW, C)
    pe_row = pe.reshape(1, HW)
    pe_col = pe.reshape(HW, 1)
    wv_row = Wv.reshape(1, C)
    g_row = gamma.reshape(1, MID)
    b_row = beta.reshape(1, MID)

    nt1 = HW // _S1
    ca = pl.pallas_call(
        _pool_kernel,
        grid=(B, nt1),
        in_specs=[
            pl.BlockSpec((1, _S1, C), lambda b, t: (b, t, 0)),
            pl.BlockSpec((1, _S1), lambda b, t: (0, t)),
            pl.BlockSpec((1, C), lambda b, t: (0, 0)),
            pl.BlockSpec((C, C), lambda b, t: (0, 0)),
            pl.BlockSpec((C, MID), lambda b, t: (0, 0)),
            pl.BlockSpec((1, MID), lambda b, t: (0, 0)),
            pl.BlockSpec((1, MID), lambda b, t: (0, 0)),
            pl.BlockSpec((MID, C), lambda b, t: (0, 0)),
        ],
        out_specs=pl.BlockSpec((1, 1, C), lambda b, t: (b, 0, 0)),
        out_shape=jax.ShapeDtypeStruct((B, 1, C), jnp.float32),
        scratch_shapes=[
            pltpu.VMEM((1, C), jnp.float32),
            pltpu.SMEM((1, 1), jnp.float32),
            pltpu.SMEM((1, 1), jnp.float32),
            pltpu.SMEM((1, 1), jnp.float32),
        ],
        compiler_params=pltpu.CompilerParams(
            dimension_semantics=("parallel", "arbitrary")),
    )(xf, pe_row, wv_row, Wk, W1, g_row, b_row, W2)

    nt2 = HW // _S2
    out = pl.pallas_call(
        _apply_kernel,
        grid=(B, nt2),
        in_specs=[
            pl.BlockSpec((1, _S2, C), lambda b, t: (b, t, 0)),
            pl.BlockSpec((_S2, 1), lambda b, t: (t, 0)),
            pl.BlockSpec((1, 1, C), lambda b, t: (b, 0, 0)),
        ],
        out_specs=pl.BlockSpec((1, _S2, C), lambda b, t: (b, t, 0)),
        out_shape=jax.ShapeDtypeStruct((B, HW, C), jnp.float32),
        compiler_params=pltpu.CompilerParams(
            dimension_semantics=("parallel", "parallel")),
    )(xf, pe_col, ca)

    return out.reshape(B, H, W, C)


# speedup vs baseline: 1.9043x; 1.0009x over previous
import jax
import jax.numpy as jnp
from jax.experimental import pallas as pl
from jax.experimental.pallas import tpu as pltpu

_LN_EPS = 1e-3
_S1 = 16384
_S2 = 8192


def _pool_kernel(x_ref, pe_ref, wv_ref, wk_ref, w1_ref, g_ref, b_ref, w2_ref,
                 out_ref, acc_ref, m_ref, l_ref, peacc_ref):
    t = pl.program_id(1)
    nt = pl.num_programs(1)

    xb = x_ref[0]
    pe_row = pe_ref[...]
    wv_row = wv_ref[...]

    logits = jax.lax.dot_general(
        wv_row, xb, (((1,), (1,)), ((), ())),
        preferred_element_type=jnp.float32)
    logits = logits + pe_row * jnp.sum(wv_row, axis=-1, keepdims=True)

    tmax = jnp.max(logits, axis=-1, keepdims=True)

    @pl.when(t == 0)
    def _init():
        p = jnp.exp(logits - tmax)
        m_ref[...] = tmax
        l_ref[...] = jnp.sum(p, axis=-1, keepdims=True)
        peacc_ref[...] = jnp.sum(p * pe_row, axis=-1, keepdims=True)
        acc_ref[...] = jnp.dot(p, xb, preferred_element_type=jnp.float32)

    @pl.when(t > 0)
    def _update():
        m_old = m_ref[...]
        m_new = jnp.maximum(m_old, tmax)
        corr = jnp.exp(m_old - m_new)
        p = jnp.exp(logits - m_new)
        m_ref[...] = m_new
        l_ref[...] = l_ref[...] * corr + jnp.sum(p, axis=-1, keepdims=True)
        peacc_ref[...] = peacc_ref[...] * corr + jnp.sum(
            p * pe_row, axis=-1, keepdims=True)
        acc_ref[...] = acc_ref[...] * corr + jnp.dot(
            p, xb, preferred_element_type=jnp.float32)

    @pl.when(t == nt - 1)
    def _epilogue():
        inv_l = 1.0 / l_ref[...]
        pooled = (acc_ref[...] + peacc_ref[...]) * inv_l
        context = jnp.dot(pooled, wk_ref[...],
                          preferred_element_type=jnp.float32)
        h = jnp.dot(context, w1_ref[...],
                    preferred_element_type=jnp.float32)
        mu = jnp.mean(h, axis=-1, keepdims=True)
        var = jnp.mean(jnp.square(h - mu), axis=-1, keepdims=True)
        h = (h - mu) * jax.lax.rsqrt(var + _LN_EPS) * g_ref[...] + b_ref[...]
        h = jnp.clip(h, 0.0, 6.0)
        out_ref[0] = jnp.dot(h, w2_ref[...],
                             preferred_element_type=jnp.float32)


def _apply_kernel(x_ref, pe_ref, ca_ref, o_ref):
    o_ref[0] = x_ref[0] + pe_ref[...] + ca_ref[0]


def kernel(x, pe, Wk, Wv, W1, gamma, beta, W2):
    B, H, W, C = x.shape
    HW = H * W
    MID = W1.shape[1]

    xf = x.reshape(B, HW, C)
    pe_row = pe.reshape(1, HW)
    pe_col = pe.reshape(HW, 1)
    wv_row = Wv.reshape(1, C)
    g_row = gamma.reshape(1, MID)
    b_row = beta.reshape(1, MID)

    nt1 = HW // _S1
    ca = pl.pallas_call(
        _pool_kernel,
        grid=(B, nt1),
        in_specs=[
            pl.BlockSpec((1, _S1, C), lambda b, t: (b, t, 0)),
            pl.BlockSpec((1, _S1), lambda b, t: (0, t)),
            pl.BlockSpec((1, C), lambda b, t: (0, 0)),
            pl.BlockSpec((C, C), lambda b, t: (0, 0)),
            pl.BlockSpec((C, MID), lambda b, t: (0, 0)),
            pl.BlockSpec((1, MID), lambda b, t: (0, 0)),
            pl.BlockSpec((1, MID), lambda b, t: (0, 0)),
            pl.BlockSpec((MID, C), lambda b, t: (0, 0)),
        ],
        out_specs=pl.BlockSpec((1, 1, C), lambda b, t: (b, 0, 0)),
        out_shape=jax.ShapeDtypeStruct((B, 1, C), jnp.float32),
        scratch_shapes=[
            pltpu.VMEM((1, C), jnp.float32),
            pltpu.VMEM((1, 1), jnp.float32),
            pltpu.VMEM((1, 1), jnp.float32),
            pltpu.VMEM((1, 1), jnp.float32),
        ],
        compiler_params=pltpu.CompilerParams(
            dimension_semantics=("parallel", "arbitrary")),
    )(xf, pe_row, wv_row, Wk, W1, g_row, b_row, W2)

    nt2 = HW // _S2
    out = pl.pallas_call(
        _apply_kernel,
        grid=(B, nt2),
        in_specs=[
            pl.BlockSpec((1, _S2, C), lambda b, t: (b, t, 0)),
            pl.BlockSpec((_S2, 1), lambda b, t: (t, 0)),
            pl.BlockSpec((1, 1, C), lambda b, t: (b, 0, 0)),
        ],
        out_specs=pl.BlockSpec((1, _S2, C), lambda b, t: (b, t, 0)),
        out_shape=jax.ShapeDtypeStruct((B, HW, C), jnp.float32),
        compiler_params=pltpu.CompilerParams(
            dimension_semantics=("parallel", "parallel")),
    )(xf, pe_col, ca)

    return out.reshape(B, H, W, C)
